# baseline (speedup 1.0000x reference)
import functools

import jax
import jax.numpy as jnp
from jax import lax
from jax.experimental import pallas as pl
from jax.experimental.pallas import tpu as pltpu
from jax.experimental.pallas import tpu_sc as plsc

CHUNK = 4096


def kernel(lS_i, lS_o, W):
    del lS_o
    n_fields, batch = lS_i.shape
    _, vocab, dim = W.shape

    info = plsc.get_sparse_core_info()
    nw = info.num_cores * info.num_subcores
    d_oct = dim // 8
    nblocks = n_fields * d_oct
    tasks_per_w = (nblocks + nw - 1) // nw

    aligned_end = (vocab // 128) * 128
    sizes = [CHUNK] * (aligned_end // CHUNK)
    if aligned_end % CHUNK:
        sizes.append(aligned_end % CHUNK)
    tail_lo = vocab - 128
    n_sweeps = len(sizes) + 1

    bag_iota = lax.broadcasted_iota(jnp.int32, (n_fields, batch), 1)
    comb = lS_i * batch + bag_iota
    bounds = []
    c0 = 0
    for sz in sizes:
        bounds.append(c0)
        c0 += sz
    bounds.append(aligned_end)

    wt = jnp.transpose(W, (0, 2, 1))
    tail_t = jnp.transpose(W[:, tail_lo:, :], (0, 2, 1))
    bag_shift = batch.bit_length() - 1

    mesh = plsc.VectorSubcoreMesh(core_axis_name="c", subcore_axis_name="s")

    def make_call(f0, nf):
        nblk = nf * d_oct
        tpw = (nblk + nw - 1) // nw

        @functools.partial(
            pl.kernel,
            out_type=jax.ShapeDtypeStruct((nblk * 8, batch), jnp.float32),
            mesh=mesh,
            compiler_params=pltpu.CompilerParams(needs_layout_passes=False),
            scratch_types=[
                pltpu.VMEM((2, 8, CHUNK), jnp.float32),
                pltpu.VMEM((batch + 16,), jnp.int32),
                pltpu.VMEM((8, batch), jnp.float32),
                pltpu.SemaphoreType.DMA,
            ],
        )
        def sc_kernel(scomb_hbm, table_hbm, tail_hbm, out_hbm,
                      buf2, scomb_v, acc, sem):
            wid = lax.axis_index("s") * info.num_cores + lax.axis_index("c")

            def task(t, _):
                block = wid + t * nw

                @pl.when(block < nblk)
                def _():
                    f = f0 + lax.shift_right_logical(block, 2)
                    db = lax.bitwise_and(block, 3)
                    f_local = lax.shift_right_logical(block, 2)
                    pltpu.sync_copy(
                        scomb_hbm.at[pl.ds(f_local * batch, batch)],
                        scomb_v.at[pl.ds(0, batch)],
                    )

                    def stage(c, slot):
                        if c < len(sizes):
                            return pltpu.async_copy(
                                table_hbm.at[
                                    f, pl.ds(db * 8, 8),
                                    pl.ds(bounds[c], sizes[c]),
                                ],
                                buf2.at[slot, :, pl.ds(0, sizes[c])],
                                sem,
                            )
                        return pltpu.async_copy(
                            tail_hbm.at[f, pl.ds(db * 8, 8), :],
                            buf2.at[slot, :, pl.ds(0, 128)],
                            sem,
                        )

                    def sweep(c, slot, p0):
                        base = bounds[c] if c < len(sizes) else tail_lo
                        sz = sizes[c] if c < len(sizes) else 128
                        lo_cut = bounds[c] * batch
                        hi_cut = (
                            (bounds[c] + sizes[c]) * batch
                            if c < len(sizes) else jnp.int32(2**31 - 1)
                        )
                        src = buf2.at[slot, :, pl.ds(0, sz)]

                        def cond(carry):
                            return carry[1]

                        def body(carry):
                            p, _ = carry
                            cv = scomb_v[pl.ds(p, 16)]
                            below = cv < hi_cut
                            msk = jnp.logical_and(cv >= lo_cut, below)
                            iv = lax.shift_right_logical(cv, bag_shift)
                            bagv = lax.bitwise_and(cv, batch - 1)
                            loc = jnp.clip(iv - base, 0, sz - 1)
                            for d in range(8):
                                dv = jnp.full((16,), d, jnp.int32)
                                v = plsc.load_gather(src, [dv, loc])
                                plsc.store_scatter(
                                    acc, [dv, bagv], v, mask=msk
                                )
                            all_in = plsc.all_reduce_population_count(
                                below
                            )[0]
                            p_new = jnp.where(all_in == 16, p + 16, p)
                            cont = jnp.logical_and(
                                all_in == 16, p_new < batch
                            )
                            return (p_new, cont)

                        p_end, _ = lax.while_loop(
                            cond, body, (p0, p0 < batch)
                        )
                        return p_end

                    handles = [stage(0, 0)]
                    pos = jnp.int32(0)
                    for c in range(n_sweeps):
                        handles[c].wait()
                        if c + 1 < n_sweeps:
                            handles.append(stage(c + 1, (c + 1) & 1))
                        pos = sweep(c, c & 1, pos)

                    pltpu.sync_copy(acc, out_hbm.at[pl.ds(block * 8, 8), :])

                return 0

            lax.fori_loop(0, tpw, task, 0)

        return sc_kernel

    splits = [(0, 2), (2, 8), (10, 8), (18, n_fields - 18)]
    parts = []
    for f0, nf in splits:
        s = lax.sort(comb[f0:f0 + nf], dimension=1).reshape(nf * batch)
        o = make_call(f0, nf)(s, wt, tail_t)
        parts.append(
            jnp.transpose(o.reshape(nf, dim, batch), (2, 0, 1))
        )
    return jnp.concatenate(parts, axis=1)


# speedup vs baseline: 1.0183x; 1.0183x over previous
import functools

import jax
import jax.numpy as jnp
from jax import lax
from jax.experimental import pallas as pl
from jax.experimental.pallas import tpu as pltpu
from jax.experimental.pallas import tpu_sc as plsc

CHUNK = 4096


def kernel(lS_i, lS_o, W):
    del lS_o
    n_fields, batch = lS_i.shape
    _, vocab, dim = W.shape

    info = plsc.get_sparse_core_info()
    nw = info.num_cores * info.num_subcores
    d_oct = dim // 8
    nblocks = n_fields * d_oct
    tasks_per_w = (nblocks + nw - 1) // nw

    aligned_end = (vocab // 128) * 128
    sizes = [CHUNK] * (aligned_end // CHUNK)
    if aligned_end % CHUNK:
        sizes.append(aligned_end % CHUNK)
    tail_lo = vocab - 128
    n_sweeps = len(sizes) + 1

    bag_iota = lax.broadcasted_iota(jnp.int32, (n_fields, batch), 1)
    comb = lS_i * batch + bag_iota
    bounds = []
    c0 = 0
    for sz in sizes:
        bounds.append(c0)
        c0 += sz
    bounds.append(aligned_end)

    wt = jnp.transpose(W, (0, 2, 1))
    tail_t = jnp.transpose(W[:, tail_lo:, :], (0, 2, 1))
    bag_shift = batch.bit_length() - 1

    mesh = plsc.VectorSubcoreMesh(core_axis_name="c", subcore_axis_name="s")

    def make_call(f0, nf):
        nblk = nf * d_oct
        tpw = (nblk + nw - 1) // nw

        @functools.partial(
            pl.kernel,
            out_type=jax.ShapeDtypeStruct((nblk * 8, batch), jnp.float32),
            mesh=mesh,
            compiler_params=pltpu.CompilerParams(needs_layout_passes=False),
            scratch_types=[
                pltpu.VMEM((2, 8, CHUNK), jnp.float32),
                pltpu.VMEM((batch + 16,), jnp.int32),
                pltpu.VMEM((8, batch), jnp.float32),
                pltpu.SemaphoreType.DMA,
            ],
        )
        def sc_kernel(scomb_hbm, table_hbm, tail_hbm, out_hbm,
                      buf2, scomb_v, acc, sem):
            wid = lax.axis_index("s") * info.num_cores + lax.axis_index("c")

            def task(t, _):
                block = wid + t * nw

                @pl.when(block < nblk)
                def _():
                    f = f0 + lax.shift_right_logical(block, 2)
                    db = lax.bitwise_and(block, 3)
                    f_local = lax.shift_right_logical(block, 2)
                    pltpu.sync_copy(
                        scomb_hbm.at[pl.ds(f_local * batch, batch)],
                        scomb_v.at[pl.ds(0, batch)],
                    )

                    def stage(c, slot):
                        if c < len(sizes):
                            return pltpu.async_copy(
                                table_hbm.at[
                                    f, pl.ds(db * 8, 8),
                                    pl.ds(bounds[c], sizes[c]),
                                ],
                                buf2.at[slot, :, pl.ds(0, sizes[c])],
                                sem,
                            )
                        return pltpu.async_copy(
                            tail_hbm.at[f, pl.ds(db * 8, 8), :],
                            buf2.at[slot, :, pl.ds(0, 128)],
                            sem,
                        )

                    def sweep(c, slot, p0):
                        base = bounds[c] if c < len(sizes) else tail_lo
                        sz = sizes[c] if c < len(sizes) else 128
                        lo_cut = bounds[c] * batch
                        hi_cut = (
                            (bounds[c] + sizes[c]) * batch
                            if c < len(sizes) else jnp.int32(2**31 - 1)
                        )
                        src = buf2.at[slot, :, pl.ds(0, sz)]

                        def cond(carry):
                            return carry[1]

                        def body(carry):
                            p, _ = carry
                            cv = scomb_v[pl.ds(p, 16)]
                            below = cv < hi_cut
                            msk = jnp.logical_and(cv >= lo_cut, below)
                            iv = lax.shift_right_logical(cv, bag_shift)
                            bagv = lax.bitwise_and(cv, batch - 1)
                            loc = jnp.clip(iv - base, 0, sz - 1)
                            for d in range(8):
                                dv = jnp.full((16,), d, jnp.int32)
                                v = plsc.load_gather(src, [dv, loc])
                                plsc.store_scatter(
                                    acc, [dv, bagv], v, mask=msk
                                )
                            all_in = plsc.all_reduce_population_count(
                                below
                            )[0]
                            p_new = jnp.where(all_in == 16, p + 16, p)
                            cont = jnp.logical_and(
                                all_in == 16, p_new < batch
                            )
                            return (p_new, cont)

                        p_end, _ = lax.while_loop(
                            cond, body, (p0, p0 < batch)
                        )
                        return p_end

                    handles = [stage(0, 0)]
                    pos = jnp.int32(0)
                    for c in range(n_sweeps):
                        handles[c].wait()
                        if c + 1 < n_sweeps:
                            handles.append(stage(c + 1, (c + 1) & 1))
                        pos = sweep(c, c & 1, pos)

                    pltpu.sync_copy(acc, out_hbm.at[pl.ds(block * 8, 8), :])

                return 0

            lax.fori_loop(0, tpw, task, 0)

        return sc_kernel

    splits = [(0, 8), (8, 8), (16, n_fields - 16)]
    parts = []
    for f0, nf in splits:
        s = lax.sort(comb[f0:f0 + nf], dimension=1).reshape(nf * batch)
        o = make_call(f0, nf)(s, wt, tail_t)
        parts.append(
            jnp.transpose(o.reshape(nf, dim, batch), (2, 0, 1))
        )
    return jnp.concatenate(parts, axis=1)
